# baseline (speedup 1.0000x reference)
import jax
import jax.numpy as jnp
from jax.experimental import pallas as pl

IGNORE_INDEX = 255
THRESHOLD = 0.7
MIN_KEPT = 100000

_BLOCK_H = 256


_ROWS = 8


def _ce_body(pred_ref, tgt_ref, loss_ref, stats_ref):
    c = pred_ref.shape[1]
    hb = pred_ref.shape[2]
    w = pred_ref.shape[3]
    nsub = hb // _ROWS

    def sub(i, carry):
        nvv, nhv, hsv = carry
        r0 = i * _ROWS
        t = tgt_ref[0, pl.ds(r0, _ROWS), :]
        m = pred_ref[0, 0, pl.ds(r0, _ROWS), :]
        for ci in range(1, c):
            m = jnp.maximum(m, pred_ref[0, ci, pl.ds(r0, _ROWS), :])
        s = jnp.zeros((_ROWS, w), jnp.float32)
        xt = jnp.zeros((_ROWS, w), jnp.float32)
        for ci in range(c):
            xc = pred_ref[0, ci, pl.ds(r0, _ROWS), :]
            s = s + jnp.exp(xc - m)
            xt = xt + jnp.where(t == ci, xc, 0.0)
        valid = t != IGNORE_INDEX
        loss = jnp.where(valid, jnp.log(s) + m - xt, -1.0)
        loss_ref[0, pl.ds(r0, _ROWS), :] = loss
        hard = loss > THRESHOLD
        nvv = nvv + valid.astype(jnp.float32)
        nhv = nhv + hard.astype(jnp.float32)
        hsv = hsv + jnp.where(hard, loss, 0.0)
        return nvv, nhv, hsv

    zeros = jnp.zeros((_ROWS, w), jnp.float32)
    nvv, nhv, hsv = jax.lax.fori_loop(0, nsub, sub, (zeros, zeros, zeros))
    stats_ref[0] = jnp.concatenate(
        [jnp.full((1, 128), jnp.sum(nvv), jnp.float32),
         jnp.full((1, 128), jnp.sum(nhv), jnp.float32),
         jnp.full((1, 128), jnp.sum(hsv), jnp.float32)], axis=0)


def _topk_mean(loss3, num_valid):
    loss_flat = loss3.reshape(-1)
    masked = jnp.where(loss_flat >= 0.0, loss_flat, -jnp.inf)
    k_static = min(MIN_KEPT, loss_flat.size)
    vals, _ = jax.lax.top_k(masked, k_static)
    k_eff = jnp.minimum(jnp.float32(MIN_KEPT), num_valid)
    keep = jnp.arange(k_static, dtype=jnp.float32) < k_eff
    s = jnp.sum(jnp.where(keep, vals, 0.0))
    return s / jnp.maximum(k_eff, 1.0)


def kernel(pred, target):
    b, c, h, w = pred.shape
    hb = min(_BLOCK_H, h)
    nh_blocks = h // hb
    grid = (b, nh_blocks)
    loss3, stats = pl.pallas_call(
        _ce_body,
        grid=grid,
        in_specs=[
            pl.BlockSpec((1, c, hb, w), lambda i, j: (i, 0, j, 0)),
            pl.BlockSpec((1, hb, w), lambda i, j: (i, j, 0)),
        ],
        out_specs=[
            pl.BlockSpec((1, hb, w), lambda i, j: (i, j, 0)),
            pl.BlockSpec((1, 3, 128), lambda i, j: (i * nh_blocks + j, 0, 0)),
        ],
        out_shape=[
            jax.ShapeDtypeStruct((b, h, w), jnp.float32),
            jax.ShapeDtypeStruct((b * nh_blocks, 3, 128), jnp.float32),
        ],
    )(pred, target)
    num_valid = jnp.sum(stats[:, 0, 0])
    num_hard = jnp.sum(stats[:, 1, 0])
    hard_sum = jnp.sum(stats[:, 2, 0])
    out = jax.lax.cond(
        num_hard < MIN_KEPT,
        lambda: _topk_mean(loss3, num_valid),
        lambda: hard_sum / jnp.maximum(num_hard, 1.0),
    )
    return jnp.where(num_valid == 0.0, jnp.float32(0.0), out)


# speedup vs baseline: 1.0900x; 1.0900x over previous
import jax
import jax.numpy as jnp
from jax.experimental import pallas as pl

IGNORE_INDEX = 255
THRESHOLD = 0.7
MIN_KEPT = 100000

_BLOCK_H = 256


_ROWS = 8


def _ce_body(pred_ref, tgt_ref, loss_ref, stats_ref):
    c = pred_ref.shape[1]
    hb = pred_ref.shape[2]
    w = pred_ref.shape[3]
    nsub = hb // _ROWS

    def sub(i, carry):
        nvv, nhv, hsv = carry
        r0 = i * _ROWS
        t = tgt_ref[0, pl.ds(r0, _ROWS), :]
        m = pred_ref[0, 0, pl.ds(r0, _ROWS), :]
        for ci in range(1, c):
            m = jnp.maximum(m, pred_ref[0, ci, pl.ds(r0, _ROWS), :])
        s = jnp.zeros((_ROWS, w), jnp.float32)
        xt = jnp.zeros((_ROWS, w), jnp.float32)
        for ci in range(c):
            xc = pred_ref[0, ci, pl.ds(r0, _ROWS), :]
            s = s + jnp.exp(xc - m)
            xt = xt + jnp.where(t == ci, xc, 0.0)
        valid = t != IGNORE_INDEX
        loss = jnp.where(valid, jnp.log(s) + m - xt, -1.0)
        loss_ref[0, pl.ds(r0, _ROWS), :] = loss
        hard = loss > THRESHOLD
        nvv = nvv + valid.astype(jnp.float32)
        nhv = nhv + hard.astype(jnp.float32)
        hsv = hsv + jnp.where(hard, loss, 0.0)
        return nvv, nhv, hsv

    zeros = jnp.zeros((_ROWS, w), jnp.float32)
    nvv, nhv, hsv = jax.lax.fori_loop(0, nsub, sub, (zeros, zeros, zeros), unroll=4)
    stats_ref[0] = jnp.concatenate(
        [jnp.full((1, 128), jnp.sum(nvv), jnp.float32),
         jnp.full((1, 128), jnp.sum(nhv), jnp.float32),
         jnp.full((1, 128), jnp.sum(hsv), jnp.float32)], axis=0)


def _topk_mean(loss3, num_valid):
    loss_flat = loss3.reshape(-1)
    masked = jnp.where(loss_flat >= 0.0, loss_flat, -jnp.inf)
    k_static = min(MIN_KEPT, loss_flat.size)
    vals, _ = jax.lax.top_k(masked, k_static)
    k_eff = jnp.minimum(jnp.float32(MIN_KEPT), num_valid)
    keep = jnp.arange(k_static, dtype=jnp.float32) < k_eff
    s = jnp.sum(jnp.where(keep, vals, 0.0))
    return s / jnp.maximum(k_eff, 1.0)


def kernel(pred, target):
    b, c, h, w = pred.shape
    hb = min(_BLOCK_H, h)
    nh_blocks = h // hb
    grid = (b, nh_blocks)
    loss3, stats = pl.pallas_call(
        _ce_body,
        grid=grid,
        in_specs=[
            pl.BlockSpec((1, c, hb, w), lambda i, j: (i, 0, j, 0)),
            pl.BlockSpec((1, hb, w), lambda i, j: (i, j, 0)),
        ],
        out_specs=[
            pl.BlockSpec((1, hb, w), lambda i, j: (i, j, 0)),
            pl.BlockSpec((1, 3, 128), lambda i, j: (i * nh_blocks + j, 0, 0)),
        ],
        out_shape=[
            jax.ShapeDtypeStruct((b, h, w), jnp.float32),
            jax.ShapeDtypeStruct((b * nh_blocks, 3, 128), jnp.float32),
        ],
    )(pred, target)
    num_valid = jnp.sum(stats[:, 0, 0])
    num_hard = jnp.sum(stats[:, 1, 0])
    hard_sum = jnp.sum(stats[:, 2, 0])
    out = jax.lax.cond(
        num_hard < MIN_KEPT,
        lambda: _topk_mean(loss3, num_valid),
        lambda: hard_sum / jnp.maximum(num_hard, 1.0),
    )
    return jnp.where(num_valid == 0.0, jnp.float32(0.0), out)


# speedup vs baseline: 1.1039x; 1.0128x over previous
import jax
import jax.numpy as jnp
from jax.experimental import pallas as pl

IGNORE_INDEX = 255
THRESHOLD = 0.7
MIN_KEPT = 100000

_BLOCK_H = 256


_ROWS = 8


def _ce_body(pred_ref, tgt_ref, loss_ref, stats_ref):
    c = pred_ref.shape[1]
    hb = pred_ref.shape[2]
    w = pred_ref.shape[3]
    nsub = hb // _ROWS

    def sub(i, carry):
        nvv, nhv, hsv = carry
        r0 = i * _ROWS
        t = tgt_ref[0, pl.ds(r0, _ROWS), :]
        m = pred_ref[0, 0, pl.ds(r0, _ROWS), :]
        for ci in range(1, c):
            m = jnp.maximum(m, pred_ref[0, ci, pl.ds(r0, _ROWS), :])
        s = jnp.zeros((_ROWS, w), jnp.float32)
        xt = jnp.zeros((_ROWS, w), jnp.float32)
        for ci in range(c):
            xc = pred_ref[0, ci, pl.ds(r0, _ROWS), :]
            s = s + jnp.exp(xc - m)
            xt = xt + jnp.where(t == ci, xc, 0.0)
        valid = t != IGNORE_INDEX
        loss = jnp.where(valid, jnp.log(s) + m - xt, -1.0)
        loss_ref[0, pl.ds(r0, _ROWS), :] = loss
        hard = loss > THRESHOLD
        nvv = nvv + valid.astype(jnp.float32)
        nhv = nhv + hard.astype(jnp.float32)
        hsv = hsv + jnp.where(hard, loss, 0.0)
        return nvv, nhv, hsv

    zeros = jnp.zeros((_ROWS, w), jnp.float32)
    nvv, nhv, hsv = jax.lax.fori_loop(0, nsub, sub, (zeros, zeros, zeros), unroll=8)
    stats_ref[0] = jnp.concatenate(
        [jnp.full((1, 128), jnp.sum(nvv), jnp.float32),
         jnp.full((1, 128), jnp.sum(nhv), jnp.float32),
         jnp.full((1, 128), jnp.sum(hsv), jnp.float32)], axis=0)


def _topk_mean(loss3, num_valid):
    loss_flat = loss3.reshape(-1)
    masked = jnp.where(loss_flat >= 0.0, loss_flat, -jnp.inf)
    k_static = min(MIN_KEPT, loss_flat.size)
    vals, _ = jax.lax.top_k(masked, k_static)
    k_eff = jnp.minimum(jnp.float32(MIN_KEPT), num_valid)
    keep = jnp.arange(k_static, dtype=jnp.float32) < k_eff
    s = jnp.sum(jnp.where(keep, vals, 0.0))
    return s / jnp.maximum(k_eff, 1.0)


def kernel(pred, target):
    b, c, h, w = pred.shape
    hb = min(_BLOCK_H, h)
    nh_blocks = h // hb
    grid = (b, nh_blocks)
    loss3, stats = pl.pallas_call(
        _ce_body,
        grid=grid,
        in_specs=[
            pl.BlockSpec((1, c, hb, w), lambda i, j: (i, 0, j, 0)),
            pl.BlockSpec((1, hb, w), lambda i, j: (i, j, 0)),
        ],
        out_specs=[
            pl.BlockSpec((1, hb, w), lambda i, j: (i, j, 0)),
            pl.BlockSpec((1, 3, 128), lambda i, j: (i * nh_blocks + j, 0, 0)),
        ],
        out_shape=[
            jax.ShapeDtypeStruct((b, h, w), jnp.float32),
            jax.ShapeDtypeStruct((b * nh_blocks, 3, 128), jnp.float32),
        ],
    )(pred, target)
    num_valid = jnp.sum(stats[:, 0, 0])
    num_hard = jnp.sum(stats[:, 1, 0])
    hard_sum = jnp.sum(stats[:, 2, 0])
    out = jax.lax.cond(
        num_hard < MIN_KEPT,
        lambda: _topk_mean(loss3, num_valid),
        lambda: hard_sum / jnp.maximum(num_hard, 1.0),
    )
    return jnp.where(num_valid == 0.0, jnp.float32(0.0), out)
